# baseline (speedup 1.0000x reference)
import jax
import jax.numpy as jnp
from jax import lax
from jax.experimental import pallas as pl
from jax.experimental.pallas import tpu as pltpu
from jax.experimental.pallas import tpu_sc as plsc

D = 2048
B = 32768
L = 16
NC, NS = 2, 16
NW = NC * NS
ROWS_PER_W = B // NW
CHUNK = 8
NCHUNK = ROWS_PER_W // CHUNK
GROUPS = D // L


def _rotor_body(x_hbm, perm_hbm, pos_hbm, out_hbm,
                perm_v, cp_v, pos_v, in_v0, in_v1, out_v0, out_v1,
                in_sem0, in_sem1, out_sem0, out_sem1):
    in_bufs = (in_v0, in_v1)
    out_bufs = (out_v0, out_v1)
    in_sems = (in_sem0, in_sem1)
    out_sems = (out_sem0, out_sem1)
    wid = lax.axis_index("s") * NC + lax.axis_index("c")
    base_row = wid * ROWS_PER_W

    pltpu.sync_copy(perm_hbm, perm_v)
    pltpu.sync_copy(pos_hbm, pos_v)
    posv = pos_v[...]
    iota = lax.iota(jnp.int32, L)

    @pl.loop(0, GROUPS)
    def _build(k):
        idx = (iota + (k * L)) + posv
        m = jnp.bitwise_and(idx, D - 1)
        cp_v[pl.ds(k * L, L)] = plsc.load_gather(perm_v, [m])

    def in_slice(c):
        return x_hbm.at[pl.ds(base_row + c * CHUNK, CHUNK)]

    def out_slice(c):
        return out_hbm.at[pl.ds(base_row + c * CHUNK, CHUNK)]

    def compute(b):
        src = in_bufs[b]
        dst = out_bufs[b]

        @plsc.parallel_loop(0, GROUPS, unroll=8)
        def _cols(k):
            cp16 = cp_v[pl.ds(k * L, L)]
            for r in range(CHUNK):
                rvec = jnp.full((L,), r, jnp.int32)
                dst[r, pl.ds(k * L, L)] = plsc.load_gather(src, [rvec, cp16])

    for b in range(2):
        pltpu.async_copy(in_slice(b), in_bufs[b], in_sems[b])

    @pl.loop(0, NCHUNK // 2)
    def _pairs(p):
        for b in range(2):
            c = p * 2 + b
            pltpu.make_async_copy(in_slice(c), in_bufs[b], in_sems[b]).wait()

            @pl.when(p >= 1)
            def _drain_prev():
                pltpu.make_async_copy(
                    out_bufs[b], out_slice(c - 2), out_sems[b]).wait()

            compute(b)
            pltpu.async_copy(out_bufs[b], out_slice(c), out_sems[b])

            @pl.when(p < NCHUNK // 2 - 1)
            def _fetch_next():
                pltpu.async_copy(in_slice(c + 2), in_bufs[b], in_sems[b])

    for b in range(2):
        pltpu.make_async_copy(
            out_bufs[b], out_slice(NCHUNK - 2 + b), out_sems[b]).wait()


def kernel(x, permutation, position):
    pos16 = jnp.broadcast_to(position.astype(jnp.int32), (L,))
    mesh = plsc.VectorSubcoreMesh(core_axis_name="c", subcore_axis_name="s")
    run = pl.kernel(
        _rotor_body,
        out_type=jax.ShapeDtypeStruct((B, D), jnp.float32),
        mesh=mesh,
        scratch_types=[
            pltpu.VMEM((D,), jnp.int32),
            pltpu.VMEM((D,), jnp.int32),
            pltpu.VMEM((L,), jnp.int32),
            pltpu.VMEM((CHUNK, D), jnp.float32),
            pltpu.VMEM((CHUNK, D), jnp.float32),
            pltpu.VMEM((CHUNK, D), jnp.float32),
            pltpu.VMEM((CHUNK, D), jnp.float32),
            pltpu.SemaphoreType.DMA,
            pltpu.SemaphoreType.DMA,
            pltpu.SemaphoreType.DMA,
            pltpu.SemaphoreType.DMA,
        ],
        compiler_params=pltpu.CompilerParams(needs_layout_passes=False),
    )
    return run(x, permutation.astype(jnp.int32), pos16)


# speedup vs baseline: 1.0311x; 1.0311x over previous
import jax
import jax.numpy as jnp
from jax import lax
from jax.experimental import pallas as pl
from jax.experimental.pallas import tpu as pltpu
from jax.experimental.pallas import tpu_sc as plsc

D = 2048
B = 32768
L = 16
NC, NS = 2, 16
NW = NC * NS
ROWS_PER_W = B // NW
CHUNK = 4
NCHUNK = ROWS_PER_W // CHUNK
GROUPS = D // L
NBUF = 4


def _rotor_body(x_hbm, perm_hbm, pos_hbm, out_hbm,
                perm_v, cp_v, pos_v, *bufs_and_sems):
    in_bufs = bufs_and_sems[0:NBUF]
    out_bufs = bufs_and_sems[NBUF:2 * NBUF]
    in_sems = bufs_and_sems[2 * NBUF:3 * NBUF]
    out_sems = bufs_and_sems[3 * NBUF:4 * NBUF]
    wid = lax.axis_index("s") * NC + lax.axis_index("c")
    base_row = wid * ROWS_PER_W

    def in_slice(c):
        return x_hbm.at[pl.ds(base_row + c * CHUNK, CHUNK)]

    def out_slice(c):
        return out_hbm.at[pl.ds(base_row + c * CHUNK, CHUNK)]

    for b in range(NBUF):
        pltpu.async_copy(in_slice(b), in_bufs[b], in_sems[b])

    pltpu.sync_copy(perm_hbm, perm_v)
    pltpu.sync_copy(pos_hbm, pos_v)
    posv = pos_v[...]
    iota = lax.iota(jnp.int32, L)

    @pl.loop(0, GROUPS)
    def _build(k):
        idx = (iota + (k * L)) + posv
        m = jnp.bitwise_and(idx, D - 1)
        cp_v[pl.ds(k * L, L)] = plsc.load_gather(perm_v, [m])

    def compute(b):
        src = in_bufs[b]
        dst = out_bufs[b]

        @plsc.parallel_loop(0, GROUPS, unroll=4)
        def _cols(k):
            cp16 = cp_v[pl.ds(k * L, L)]
            for r in range(CHUNK):
                rvec = jnp.full((L,), r, jnp.int32)
                dst[r, pl.ds(k * L, L)] = plsc.load_gather(src, [rvec, cp16])

    @pl.loop(0, NCHUNK // NBUF)
    def _rounds(p):
        for b in range(NBUF):
            c = p * NBUF + b
            pltpu.make_async_copy(in_slice(c), in_bufs[b], in_sems[b]).wait()

            @pl.when(p >= 1)
            def _drain_prev():
                pltpu.make_async_copy(
                    out_bufs[b], out_slice(c - NBUF), out_sems[b]).wait()

            compute(b)
            pltpu.async_copy(out_bufs[b], out_slice(c), out_sems[b])

            @pl.when(p < NCHUNK // NBUF - 1)
            def _fetch_next():
                pltpu.async_copy(in_slice(c + NBUF), in_bufs[b], in_sems[b])

    for b in range(NBUF):
        pltpu.make_async_copy(
            out_bufs[b], out_slice(NCHUNK - NBUF + b), out_sems[b]).wait()


def kernel(x, permutation, position):
    pos16 = jnp.broadcast_to(position.astype(jnp.int32), (L,))
    mesh = plsc.VectorSubcoreMesh(core_axis_name="c", subcore_axis_name="s")
    run = pl.kernel(
        _rotor_body,
        out_type=jax.ShapeDtypeStruct((B, D), jnp.float32),
        mesh=mesh,
        scratch_types=(
            [pltpu.VMEM((D,), jnp.int32),
             pltpu.VMEM((D,), jnp.int32),
             pltpu.VMEM((L,), jnp.int32)]
            + [pltpu.VMEM((CHUNK, D), jnp.float32)] * (2 * NBUF)
            + [pltpu.SemaphoreType.DMA] * (2 * NBUF)
        ),
        compiler_params=pltpu.CompilerParams(needs_layout_passes=False),
    )
    return run(x, permutation.astype(jnp.int32), pos16)
